# baseline (speedup 1.0000x reference)
import jax
import jax.numpy as jnp
from jax import lax
from jax.experimental import pallas as pl
from jax.experimental.pallas import tpu as pltpu

_B, _N, _C, _K = 8, 20000, 21, 200
_FG = _C - 1
_IOU_THR, _CONF_THR = 0.45, 0.01
_NCHUNK = 8
_CW = _N // _NCHUNK
_NEG = float("-inf")


def _detect_kernel(pred_ref,
                   flag_ref, conf_ref, bx_ref, by_ref, bw_ref, bh_ref,
                   fg_ref):
    for c in range(_NCHUNK):
        sl = pl.ds(c * _CW, _CW)
        logits = pred_ref[0, 4:4 + _C, sl]
        lmax = jnp.max(logits, axis=0, keepdims=True)
        e = jnp.exp(logits - lmax)
        probs = e / jnp.sum(e, axis=0, keepdims=True)
        fg_ref[:, sl] = probs[:_FG, :]

    laneK = lax.broadcasted_iota(jnp.int32, (_FG, _K), 1)

    def topk_body(k, carry):
        tv_a, bx_a, by_a, bw_a, bh_a, imin_prev = carry
        mv = jnp.full((_FG, 1), _NEG, jnp.float32)
        im = jnp.full((_FG, 1), _N, jnp.int32)
        bx = jnp.zeros((_FG, 1), jnp.float32)
        by = jnp.zeros((_FG, 1), jnp.float32)
        bw = jnp.zeros((_FG, 1), jnp.float32)
        bh = jnp.zeros((_FG, 1), jnp.float32)
        for c in range(_NCHUNK):
            sl = pl.ds(c * _CW, _CW)
            colid = lax.broadcasted_iota(jnp.int32, (_FG, _CW), 1) + c * _CW
            fgc = fg_ref[:, sl]
            fgc = jnp.where(colid == imin_prev, _NEG, fgc)
            fg_ref[:, sl] = fgc
            mc = jnp.max(fgc, axis=1, keepdims=True)
            selc = jnp.where(fgc == mc, colid, _N)
            icc = jnp.min(selc, axis=1, keepdims=True)
            oh = colid == icc
            cxc = pred_ref[0, 0:1, sl]
            cyc = pred_ref[0, 1:2, sl]
            cwc = pred_ref[0, 2:3, sl]
            chc = pred_ref[0, 3:4, sl]
            bxc = jnp.sum(jnp.where(oh, cxc, 0.0), axis=1, keepdims=True)
            byc = jnp.sum(jnp.where(oh, cyc, 0.0), axis=1, keepdims=True)
            bwc = jnp.sum(jnp.where(oh, cwc, 0.0), axis=1, keepdims=True)
            bhc = jnp.sum(jnp.where(oh, chc, 0.0), axis=1, keepdims=True)
            better = (mc > mv) | ((mc == mv) & (icc < im))
            mv = jnp.where(better, mc, mv)
            im = jnp.where(better, icc, im)
            bx = jnp.where(better, bxc, bx)
            by = jnp.where(better, byc, by)
            bw = jnp.where(better, bwc, bw)
            bh = jnp.where(better, bhc, bh)
        lm = laneK == k
        return (jnp.where(lm, mv, tv_a),
                jnp.where(lm, bx, bx_a),
                jnp.where(lm, by, by_a),
                jnp.where(lm, bw, bw_a),
                jnp.where(lm, bh, bh_a),
                im)

    init = (jnp.zeros((_FG, _K), jnp.float32),
            jnp.zeros((_FG, _K), jnp.float32),
            jnp.zeros((_FG, _K), jnp.float32),
            jnp.zeros((_FG, _K), jnp.float32),
            jnp.zeros((_FG, _K), jnp.float32),
            jnp.full((_FG, 1), -1, jnp.int32))
    tv, bx, by, bw, bh, _ = lax.fori_loop(0, _K, topk_body, init)

    x1 = bx - bw / 2.0
    y1 = by - bh / 2.0
    x2 = bx + bw / 2.0
    y2 = by + bh / 2.0
    area = jnp.maximum(x2 - x1, 0.0) * jnp.maximum(y2 - y1, 0.0)

    def nms_body(i, keep):
        lm = laneK == i

        def ext(a):
            return jnp.sum(jnp.where(lm, a, 0.0), axis=1, keepdims=True)

        xi1, yi1, xi2, yi2 = ext(x1), ext(y1), ext(x2), ext(y2)
        ai, ki = ext(area), ext(keep)
        ix1 = jnp.maximum(x1, xi1)
        iy1 = jnp.maximum(y1, yi1)
        ix2 = jnp.minimum(x2, xi2)
        iy2 = jnp.minimum(y2, yi2)
        inter = jnp.maximum(ix2 - ix1, 0.0) * jnp.maximum(iy2 - iy1, 0.0)
        union = area + ai - inter
        iou = inter / jnp.maximum(union, 1e-8)
        sup = (iou > _IOU_THR) & (laneK > i) & (ki > 0.5)
        return jnp.where(sup, 0.0, keep)

    keep = lax.fori_loop(0, _K, nms_body, jnp.ones((_FG, _K), jnp.float32))
    keep = keep * (tv > _CONF_THR).astype(jnp.float32)

    cid = lax.broadcasted_iota(jnp.int32, (_FG, _K), 0).astype(jnp.float32)
    flag_ref[0] = cid * keep
    conf_ref[0] = tv * keep
    bx_ref[0] = bx * keep
    by_ref[0] = by * keep
    bw_ref[0] = bw * keep
    bh_ref[0] = bh * keep


def kernel(predicts):
    pt = jnp.transpose(predicts, (0, 2, 1))
    outs = pl.pallas_call(
        _detect_kernel,
        grid=(_B,),
        in_specs=[pl.BlockSpec((1, 4 + _C, _N), lambda b: (b, 0, 0))],
        out_specs=[pl.BlockSpec((1, _FG, _K), lambda b: (b, 0, 0))
                   for _ in range(6)],
        out_shape=[jax.ShapeDtypeStruct((_B, _FG, _K), jnp.float32)
                   for _ in range(6)],
        scratch_shapes=[pltpu.VMEM((_FG, _N), jnp.float32)],
    )(pt)
    return jnp.stack(outs, axis=-1)
